# baseline (speedup 1.0000x reference)
import functools

import jax
import jax.numpy as jnp
import numpy as np
from jax import lax
from jax.experimental import pallas as pl
from jax.experimental.pallas import tpu as pltpu
from jax.experimental.pallas import tpu_sc as plsc

_B = 32
_C = 384
_G = 96
_GROUPBY = 4
_NSEL = 47

_LT = _C // 128
_ROWS_PER_B = 9408
_UNIT = 24
_CHUNK_ROWS = 336
_NCHUNK = _ROWS_PER_B // _CHUNK_ROWS


def _group_mask():
    key = jax.random.key(42)
    keys = jax.random.split(key, _B)
    notp = jnp.arange(1, _G, dtype=jnp.int32)
    chosen = jax.vmap(lambda k: jax.random.permutation(k, notp)[:_NSEL])(keys)
    mask = jnp.zeros((_B, _G), jnp.float32).at[:, 0].set(1.0)
    mask = mask.at[jnp.arange(_B)[:, None], chosen].set(1.0)
    return mask


_MASK_BC = np.asarray(
    jax.device_get(jnp.repeat(_group_mask(), _GROUPBY, axis=1))
)


def _sc_body(x_hbm, m_hbm, o_hbm, buf, mbuf, gsem, ssem):
    info = plsc.get_sparse_core_info()
    nc = info.num_cores
    b = lax.axis_index("s") * nc + lax.axis_index("c")
    pltpu.sync_copy(m_hbm.at[b], mbuf)
    base = b * _ROWS_PER_B

    def gather(ci, s):
        return pltpu.make_async_copy(
            x_hbm.at[pl.ds(base + ci * _CHUNK_ROWS, _CHUNK_ROWS)],
            buf.at[s], gsem.at[s])

    def scatter(ci, s):
        return pltpu.make_async_copy(
            buf.at[s],
            o_hbm.at[pl.ds(base + ci * _CHUNK_ROWS, _CHUNK_ROWS)],
            ssem.at[s])

    def compute(s):
        for lt in range(_LT):
            for j in range(128 // 16):
                mv = mbuf[pl.ds(lt * 128 + j * 16, 16)]

                def unit(u, mv):
                    for wi in range(8):
                        r = u * _UNIT + lt * 8 + wi
                        sl = (s, r, pl.ds(j * 16, 16))
                        buf[sl] = buf[sl] * mv
                    return mv

                lax.fori_loop(0, _CHUNK_ROWS // _UNIT, unit, mv)

    gather(0, 0).start()
    gather(1, 1).start()

    def pair(p, carry):
        a = 2 * p
        gather(a, 0).wait()
        compute(0)
        scatter(a, 0).start()

        @pl.when(p < _NCHUNK // 2 - 1)
        def _():
            scatter(a, 0).wait()
            gather(a + 2, 0).start()

        gather(a + 1, 1).wait()
        compute(1)
        scatter(a + 1, 1).start()

        @pl.when(p < _NCHUNK // 2 - 1)
        def _():
            scatter(a + 1, 1).wait()
            gather(a + 3, 1).start()

        return carry

    lax.fori_loop(0, _NCHUNK // 2, pair, 0)
    scatter(_ROWS_PER_B // _CHUNK_ROWS - 2, 0).wait()
    scatter(_ROWS_PER_B // _CHUNK_ROWS - 1, 1).wait()


def kernel(input):
    B, C, H, W = input.shape
    rows = B * H * (W // 8) * _LT * 8
    x2 = (
        input.reshape(B, _LT, 128, H, W // 8, 8)
        .transpose(0, 3, 4, 1, 5, 2)
        .reshape(rows, 128)
    )
    m = jnp.asarray(_MASK_BC)
    mesh = plsc.VectorSubcoreMesh(core_axis_name="c", subcore_axis_name="s")
    sc_call = functools.partial(
        pl.kernel,
        mesh=mesh,
        out_type=jax.ShapeDtypeStruct((rows, 128), jnp.float32),
        scratch_types=[
            pltpu.VMEM((2, _CHUNK_ROWS, 128), jnp.float32),
            pltpu.VMEM((_C,), jnp.float32),
            pltpu.SemaphoreType.DMA((2,)),
            pltpu.SemaphoreType.DMA((2,)),
        ],
    )(_sc_body)
    out = sc_call(x2, m)
    return (
        out.reshape(B, H, W // 8, _LT, 8, 128)
        .transpose(0, 3, 5, 1, 2, 4)
        .reshape(B, C, H, W)
    )
